# baseline (speedup 1.0000x reference)
import functools

import jax
import jax.numpy as jnp
from jax import lax
from jax.experimental import pallas as pl
from jax.experimental.pallas import tpu as pltpu
from jax.experimental.pallas import tpu_sc as plsc

N = 10000
E = 320000
D = 128
D2 = D // 2

NC = 2
NS = 16

EPT = E // NS
C = 80
BLK = 2000


def _tc_pre_body(x_ref, w_ref, att2_ref, h_ref, asd_ref):
    h = jnp.dot(x_ref[...], w_ref[...], preferred_element_type=jnp.float32)
    h_ref[...] = h
    asd_ref[...] = jnp.dot(h, att2_ref[...], preferred_element_type=jnp.float32)


def _tc_pre(x, W, att2):
    return pl.pallas_call(
        _tc_pre_body,
        out_shape=[
            jax.ShapeDtypeStruct((N, D), jnp.float32),
            jax.ShapeDtypeStruct((N, 2), jnp.float32),
        ],
    )(x, W, att2)


def _sc_agg_body(ht_hbm, asrc_hbm, adst_hbm, src_hbm, dst_hbm,
                 S0_out, S1_out, den_out,
                 sblk_v, dblk_v, sidx_v, didx_v, asrc_v, adst_v, ee_v,
                 rowsA_v, S_sh, den_sh, sem, sem2):
    c = lax.axis_index("c")
    s = lax.axis_index("s")

    z16f = jnp.zeros((16,), jnp.float32)

    def zrow(r, carry):
        for k in range(D2 // 16):
            rowsA_v[r, pl.ds(k * 16, 16)] = z16f
        return carry
    lax.fori_loop(0, C, zrow, 0)
    for k in range(C // 16):
        ee_v[pl.ds(k * 16, 16)] = z16f

    dstart = jnp.minimum(s * 640, N - 640)
    for j in range(640 // C):
        pltpu.sync_copy(rowsA_v, S_sh.at[pl.ds(dstart + j * C, C)])
        pltpu.sync_copy(ee_v, den_sh.at[pl.ds(dstart + j * C, C)])

    plsc.subcore_barrier()

    def chunk(j, carry):
        off = j * C
        for k in range(C // 16):
            s16 = sblk_v[pl.ds(off + k * 16, 16)]
            sidx_v[pl.ds(k * 16, 16)] = s16 * 2 + c
            didx_v[pl.ds(k * 16, 16)] = dblk_v[pl.ds(off + k * 16, 16)]
        cp = pltpu.async_copy(ht_hbm.at[sidx_v], rowsA_v, sem)
        cpa = pltpu.async_copy(asrc_hbm.at[sblk_v.at[pl.ds(off, C)]],
                               asrc_v, sem2)
        cpd = pltpu.async_copy(adst_hbm.at[didx_v], adst_v, sem2)
        cpa.wait()
        cpd.wait()
        for k in range(C // 16):
            e = asrc_v[pl.ds(k * 16, 16)] + adst_v[pl.ds(k * 16, 16)]
            e = jnp.where(e >= 0.0, e, 0.2 * e)
            ee_v[pl.ds(k * 16, 16)] = jnp.exp(e)
        cp.wait()

        def srow(r, carry2):
            eb = plsc.load_gather(ee_v, [jnp.full((16,), r, jnp.int32)])
            for k in range(D2 // 16):
                rowsA_v[r, pl.ds(k * 16, 16)] = rowsA_v[r, pl.ds(k * 16, 16)] * eb
            return carry2
        lax.fori_loop(0, C, srow, 0)

        @pl.when(c == 0)
        def _():
            pltpu.sync_copy(ee_v, den_sh.at[didx_v], add=True)
        pltpu.sync_copy(rowsA_v, S_sh.at[didx_v], add=True)
        return carry

    def block(b, carry):
        base = s * EPT + b * BLK
        pltpu.sync_copy(src_hbm.at[pl.ds(base, BLK)], sblk_v)
        pltpu.sync_copy(dst_hbm.at[pl.ds(base, BLK)], dblk_v)
        lax.fori_loop(0, BLK // C, chunk, carry)
        return carry
    lax.fori_loop(0, EPT // BLK, block, 0)

    plsc.subcore_barrier()

    def strip_writes(lo, n):
        @pl.when(c == 0)
        def _():
            pltpu.sync_copy(S_sh.at[pl.ds(lo, n)], S0_out.at[pl.ds(lo, n)])
            pltpu.sync_copy(den_sh.at[pl.ds(lo, n)], den_out.at[pl.ds(lo, n)])

        @pl.when(c == 1)
        def _():
            pltpu.sync_copy(S_sh.at[pl.ds(lo, n)], S1_out.at[pl.ds(lo, n)])

    @pl.when(s < NS - 1)
    def _():
        strip_writes(s * 624, 624)

    @pl.when(s == NS - 1)
    def _():
        strip_writes(N - 640, 640)


_sc_agg = functools.partial(
    pl.kernel,
    out_type=[
        jax.ShapeDtypeStruct((N, D2), jnp.float32),
        jax.ShapeDtypeStruct((N, D2), jnp.float32),
        jax.ShapeDtypeStruct((N,), jnp.float32),
    ],
    mesh=plsc.VectorSubcoreMesh(
        core_axis_name="c", subcore_axis_name="s",
        num_cores=NC, num_subcores=NS),
    scratch_types=[
        pltpu.VMEM((BLK,), jnp.int32),
        pltpu.VMEM((BLK,), jnp.int32),
        pltpu.VMEM((C,), jnp.int32),
        pltpu.VMEM((C,), jnp.int32),
        pltpu.VMEM((C,), jnp.float32),
        pltpu.VMEM((C,), jnp.float32),
        pltpu.VMEM((C,), jnp.float32),
        pltpu.VMEM((C, D2), jnp.float32),
        pltpu.VMEM_SHARED((N, D2), jnp.float32),
        pltpu.VMEM_SHARED((N,), jnp.float32),
        pltpu.SemaphoreType.DMA,
        pltpu.SemaphoreType.DMA,
    ],
    compiler_params=pltpu.CompilerParams(
        use_tc_tiling_on_sc=False, needs_layout_passes=False),
)(_sc_agg_body)


def _tc_post_body(S0_ref, S1_ref, den_ref, bias_ref, out_ref):
    den = den_ref[...] + 1e-16
    out_ref[:, :D2] = jnp.tanh(S0_ref[...] / den + bias_ref[:, :D2])
    out_ref[:, D2:] = jnp.tanh(S1_ref[...] / den + bias_ref[:, D2:])


def _tc_post(S0, S1, den_col, bias2):
    return pl.pallas_call(
        _tc_post_body,
        out_shape=jax.ShapeDtypeStruct((N, D), jnp.float32),
    )(S0, S1, den_col, bias2)


def kernel(x, edge_index, W, att_src, att_dst, bias):
    src = edge_index[0].astype(jnp.int32)
    dst = edge_index[1].astype(jnp.int32)
    att2 = jnp.stack([att_src, att_dst], axis=1)
    h, asd = _tc_pre(x, W, att2)
    ht = h.reshape(2 * N, D2)
    S0, S1, den_p = _sc_agg(ht, asd[:, 0], asd[:, 1], src, dst)
    return _tc_post(S0, S1, den_p.reshape(N, 1), bias.reshape(1, D))


# speedup vs baseline: 1.5534x; 1.5534x over previous
import functools

import jax
import jax.numpy as jnp
from jax import lax
from jax.experimental import pallas as pl
from jax.experimental.pallas import tpu as pltpu
from jax.experimental.pallas import tpu_sc as plsc

N = 10000
E = 320000
D = 128
D2 = D // 2

NC = 2
NS = 16

EPT = E // NS
C = 80
BLK = 2000


def _tc_pre_body(x_ref, w_ref, att2_ref, h_ref, asd_ref):
    h = jnp.dot(x_ref[...], w_ref[...], preferred_element_type=jnp.float32)
    h_ref[...] = h
    asd_ref[...] = jnp.dot(h, att2_ref[...], preferred_element_type=jnp.float32)


def _tc_pre(x, W, att2):
    return pl.pallas_call(
        _tc_pre_body,
        out_shape=[
            jax.ShapeDtypeStruct((N, D), jnp.float32),
            jax.ShapeDtypeStruct((N, 2), jnp.float32),
        ],
    )(x, W, att2)


def _sc_agg_body(ht_hbm, asrc_hbm, adst_hbm, src_hbm, dst_hbm,
                 S0_out, S1_out, den_out,
                 sblk_v, dblk_v,
                 sidxA_v, didxA_v, asrcA_v, adstA_v, rowsA_v,
                 sidxB_v, didxB_v, asrcB_v, adstB_v, rowsB_v,
                 ee_v, S_sh, den_sh, semHA, semAA, semHB, semAB):
    c = lax.axis_index("c")
    s = lax.axis_index("s")

    z16f = jnp.zeros((16,), jnp.float32)

    def zrow(r, carry):
        for k in range(D2 // 16):
            rowsA_v[r, pl.ds(k * 16, 16)] = z16f
        return carry
    lax.fori_loop(0, C, zrow, 0)
    for k in range(C // 16):
        ee_v[pl.ds(k * 16, 16)] = z16f

    dstart = jnp.minimum(s * 640, N - 640)
    for j in range(640 // C):
        pltpu.sync_copy(rowsA_v, S_sh.at[pl.ds(dstart + j * C, C)])
        pltpu.sync_copy(ee_v, den_sh.at[pl.ds(dstart + j * C, C)])

    plsc.subcore_barrier()

    def issue(j, sidx, didx, asrc, adst, rows, hs, asm):
        off = j * C
        for k in range(C // 16):
            s16 = sblk_v[pl.ds(off + k * 16, 16)]
            sidx[pl.ds(k * 16, 16)] = s16 * 2 + c
            didx[pl.ds(k * 16, 16)] = dblk_v[pl.ds(off + k * 16, 16)]
        pltpu.async_copy(ht_hbm.at[sidx], rows, hs)
        pltpu.async_copy(asrc_hbm.at[sblk_v.at[pl.ds(off, C)]], asrc, asm)
        pltpu.async_copy(adst_hbm.at[didx], adst, asm)

    def drain(sidx, didx, asrc, adst, rows, hs, asm):
        pltpu.make_async_copy(asrc_hbm.at[sblk_v.at[pl.ds(0, C)]],
                              asrc, asm).wait()
        pltpu.make_async_copy(adst_hbm.at[didx], adst, asm).wait()
        for k in range(C // 16):
            e = asrc[pl.ds(k * 16, 16)] + adst[pl.ds(k * 16, 16)]
            e = jnp.where(e >= 0.0, e, 0.2 * e)
            ee_v[pl.ds(k * 16, 16)] = jnp.exp(e)
        pltpu.make_async_copy(ht_hbm.at[sidx], rows, hs).wait()

        def srow(r, carry2):
            eb = plsc.load_gather(ee_v, [jnp.full((16,), r, jnp.int32)])
            for k in range(D2 // 16):
                rows[r, pl.ds(k * 16, 16)] = rows[r, pl.ds(k * 16, 16)] * eb
            return carry2
        lax.fori_loop(0, C, srow, 0)

        @pl.when(c == 0)
        def _():
            pltpu.sync_copy(ee_v, den_sh.at[didx], add=True)
        pltpu.sync_copy(rows, S_sh.at[didx], add=True)

    def issueA(j):
        issue(j, sidxA_v, didxA_v, asrcA_v, adstA_v, rowsA_v, semHA, semAA)

    def issueB(j):
        issue(j, sidxB_v, didxB_v, asrcB_v, adstB_v, rowsB_v, semHB, semAB)

    def drainA(j):
        drain(sidxA_v, didxA_v, asrcA_v, adstA_v, rowsA_v, semHA, semAA)

    def drainB(j):
        drain(sidxB_v, didxB_v, asrcB_v, adstB_v, rowsB_v, semHB, semAB)

    CPB = BLK // C

    def block(b, carry):
        base = s * EPT + b * BLK
        pltpu.sync_copy(src_hbm.at[pl.ds(base, BLK)], sblk_v)
        pltpu.sync_copy(dst_hbm.at[pl.ds(base, BLK)], dblk_v)
        issueA(0)

        def pair(i, carry2):
            issueB(2 * i + 1)
            drainA(2 * i)
            issueA(2 * i + 2)
            drainB(2 * i + 1)
            return carry2
        lax.fori_loop(0, (CPB - 3) // 2, pair, 0)
        issueB(CPB - 2)
        drainA(CPB - 3)
        issueA(CPB - 1)
        drainB(CPB - 2)
        drainA(CPB - 1)
        return carry
    lax.fori_loop(0, EPT // BLK, block, 0)

    plsc.subcore_barrier()

    def strip_writes(lo, n):
        @pl.when(c == 0)
        def _():
            pltpu.sync_copy(S_sh.at[pl.ds(lo, n)], S0_out.at[pl.ds(lo, n)])
            pltpu.sync_copy(den_sh.at[pl.ds(lo, n)], den_out.at[pl.ds(lo, n)])

        @pl.when(c == 1)
        def _():
            pltpu.sync_copy(S_sh.at[pl.ds(lo, n)], S1_out.at[pl.ds(lo, n)])

    @pl.when(s < NS - 1)
    def _():
        strip_writes(s * 624, 624)

    @pl.when(s == NS - 1)
    def _():
        strip_writes(N - 640, 640)


_sc_agg = functools.partial(
    pl.kernel,
    out_type=[
        jax.ShapeDtypeStruct((N, D2), jnp.float32),
        jax.ShapeDtypeStruct((N, D2), jnp.float32),
        jax.ShapeDtypeStruct((N,), jnp.float32),
    ],
    mesh=plsc.VectorSubcoreMesh(
        core_axis_name="c", subcore_axis_name="s",
        num_cores=NC, num_subcores=NS),
    scratch_types=[
        pltpu.VMEM((BLK,), jnp.int32),
        pltpu.VMEM((BLK,), jnp.int32),
        pltpu.VMEM((C,), jnp.int32),
        pltpu.VMEM((C,), jnp.int32),
        pltpu.VMEM((C,), jnp.float32),
        pltpu.VMEM((C,), jnp.float32),
        pltpu.VMEM((C, D2), jnp.float32),
        pltpu.VMEM((C,), jnp.int32),
        pltpu.VMEM((C,), jnp.int32),
        pltpu.VMEM((C,), jnp.float32),
        pltpu.VMEM((C,), jnp.float32),
        pltpu.VMEM((C, D2), jnp.float32),
        pltpu.VMEM((C,), jnp.float32),
        pltpu.VMEM_SHARED((N, D2), jnp.float32),
        pltpu.VMEM_SHARED((N,), jnp.float32),
        pltpu.SemaphoreType.DMA,
        pltpu.SemaphoreType.DMA,
        pltpu.SemaphoreType.DMA,
        pltpu.SemaphoreType.DMA,
    ],
    compiler_params=pltpu.CompilerParams(
        use_tc_tiling_on_sc=False, needs_layout_passes=False),
)(_sc_agg_body)


def _tc_post_body(S0_ref, S1_ref, den_ref, bias_ref, out_ref):
    den = den_ref[...] + 1e-16
    out_ref[:, :D2] = jnp.tanh(S0_ref[...] / den + bias_ref[:, :D2])
    out_ref[:, D2:] = jnp.tanh(S1_ref[...] / den + bias_ref[:, D2:])


def _tc_post(S0, S1, den_col, bias2):
    return pl.pallas_call(
        _tc_post_body,
        out_shape=jax.ShapeDtypeStruct((N, D), jnp.float32),
    )(S0, S1, den_col, bias2)


def kernel(x, edge_index, W, att_src, att_dst, bias):
    src = edge_index[0].astype(jnp.int32)
    dst = edge_index[1].astype(jnp.int32)
    att2 = jnp.stack([att_src, att_dst], axis=1)
    h, asd = _tc_pre(x, W, att2)
    ht = h.reshape(2 * N, D2)
    S0, S1, den_p = _sc_agg(ht, asd[:, 0], asd[:, 1], src, dst)
    return _tc_post(S0, S1, den_p.reshape(N, 1), bias.reshape(1, D))
